# baseline (speedup 1.0000x reference)
import functools

import jax
import jax.numpy as jnp
from jax import lax
from jax.experimental import pallas as pl
from jax.experimental.pallas import tpu as pltpu
from jax.experimental.pallas import tpu_sc as plsc

N = 10000
K = 32
D_IN = 128
D_OUT = 128

BN = 400
R_TC = 6400
R_SC = N - R_TC

SC_NC = 2
SC_NS = 16
SC_NW = SC_NC * SC_NS
CH = 8
NCH_G = R_SC // CH
LANES = 16
HREG = D_IN // LANES


def _head_body(self_ref, neigh_ref, wts_ref, wtn_ref, b_ref, out_ref):
    neigh_mean = jnp.sum(neigh_ref[...], axis=1) * (1.0 / K)
    t = jnp.dot(self_ref[...], wts_ref[...], preferred_element_type=jnp.float32)
    t = t + jnp.dot(neigh_mean, wtn_ref[...], preferred_element_type=jnp.float32)
    t = t + b_ref[...]
    c = jnp.maximum(t, 0.0)
    norm2 = jnp.sum(c * c, axis=1, keepdims=True)
    out_ref[...] = c * jax.lax.rsqrt(jnp.maximum(norm2, 1e-24))


def _tail_body(self_ref, scsum_ref, wts_ref, wtn_ref, b_ref, out_ref):
    neigh_mean = scsum_ref[...] * (1.0 / K)
    t = jnp.dot(self_ref[...], wts_ref[...], preferred_element_type=jnp.float32)
    t = t + jnp.dot(neigh_mean, wtn_ref[...], preferred_element_type=jnp.float32)
    t = t + b_ref[...]
    c = jnp.maximum(t, 0.0)
    norm2 = jnp.sum(c * c, axis=1, keepdims=True)
    out_ref[...] = c * jax.lax.rsqrt(jnp.maximum(norm2, 1e-24))


def _sc_body(neigh_hbm, out_hbm, buf, obuf, sem_in, sem_out):
    ci = lax.axis_index("c")
    si = lax.axis_index("s")
    w = si * SC_NC + ci
    nj = (NCH_G - 1 - w) // SC_NW + 1

    def step(j, carry):
        chunk = w + j * SC_NW
        row = R_TC + chunk * CH
        pltpu.async_copy(neigh_hbm.at[pl.ds(row, CH)], buf, sem_in).wait()
        for i in range(CH):
            def kbody(k8, accs):
                new = list(accs)
                for kk in range(8):
                    base = (k8 * 8 + kk) * D_IN
                    for h in range(HREG):
                        new[h] = new[h] + buf[i, pl.ds(base + h * LANES, LANES)]
                return tuple(new)
            accs = lax.fori_loop(
                0, K // 8, kbody,
                tuple(jnp.zeros((LANES,), jnp.float32) for _ in range(HREG)),
            )
            for h in range(HREG):
                obuf[i, pl.ds(h * LANES, LANES)] = accs[h]
        pltpu.async_copy(obuf, out_hbm.at[pl.ds(chunk * CH, CH)], sem_out).wait()
        return carry

    lax.fori_loop(0, nj, step, 0)


_sc_neigh_sum = functools.partial(
    pl.kernel,
    out_type=jax.ShapeDtypeStruct((R_SC, D_IN), jnp.float32),
    mesh=plsc.VectorSubcoreMesh(
        core_axis_name="c", subcore_axis_name="s",
        num_cores=SC_NC, num_subcores=SC_NS,
    ),
    scratch_types=[
        pltpu.VMEM((CH, K * D_IN), jnp.float32),
        pltpu.VMEM((CH, D_IN), jnp.float32),
        pltpu.SemaphoreType.DMA,
        pltpu.SemaphoreType.DMA,
    ],
)(_sc_body)


def kernel(self_embs, neigh_embs, W_self, b_self, W_neigh, b_neigh):
    wts = W_self.T
    wtn = W_neigh.T
    b = (b_self + b_neigh).reshape(1, D_OUT)
    neigh2d = neigh_embs.reshape(N, K * D_IN)

    sc_sum = _sc_neigh_sum(neigh2d)

    head = pl.pallas_call(
        _head_body,
        grid=(R_TC // BN,),
        in_specs=[
            pl.BlockSpec((BN, D_IN), lambda i: (i, 0)),
            pl.BlockSpec((BN, K, D_IN), lambda i: (i, 0, 0)),
            pl.BlockSpec((D_IN, D_OUT), lambda i: (0, 0)),
            pl.BlockSpec((D_IN, D_OUT), lambda i: (0, 0)),
            pl.BlockSpec((1, D_OUT), lambda i: (0, 0)),
        ],
        out_specs=pl.BlockSpec((BN, D_OUT), lambda i: (i, 0)),
        out_shape=jax.ShapeDtypeStruct((R_TC, D_OUT), jnp.float32),
        compiler_params=pltpu.CompilerParams(
            dimension_semantics=("arbitrary",),
        ),
    )(self_embs, neigh_embs, wts, wtn, b)

    nhead = R_TC // BN
    tail = pl.pallas_call(
        _tail_body,
        grid=(R_SC // BN,),
        in_specs=[
            pl.BlockSpec((BN, D_IN), lambda i: (i + nhead, 0)),
            pl.BlockSpec((BN, D_IN), lambda i: (i, 0)),
            pl.BlockSpec((D_IN, D_OUT), lambda i: (0, 0)),
            pl.BlockSpec((D_IN, D_OUT), lambda i: (0, 0)),
            pl.BlockSpec((1, D_OUT), lambda i: (0, 0)),
        ],
        out_specs=pl.BlockSpec((BN, D_OUT), lambda i: (i, 0)),
        out_shape=jax.ShapeDtypeStruct((R_SC, D_OUT), jnp.float32),
        compiler_params=pltpu.CompilerParams(
            dimension_semantics=("arbitrary",),
        ),
    )(self_embs, sc_sum, wts, wtn, b)

    return jnp.concatenate([head, tail], axis=0)


# speedup vs baseline: 2.2563x; 2.2563x over previous
import functools

import jax
import jax.numpy as jnp
from jax import lax
from jax.experimental import pallas as pl
from jax.experimental.pallas import tpu as pltpu
from jax.experimental.pallas import tpu_sc as plsc

N = 10000
K = 32
D_IN = 128
D_OUT = 128

BN = 400
R_TC = 6400
R_SC = N - R_TC

SC_NC = 2
SC_NS = 16
SC_NW = SC_NC * SC_NS
CH = 8
NCH_G = R_SC // CH
LANES = 16
HREG = D_IN // LANES


def _head_body(self_ref, neigh_ref, wts_ref, wtn_ref, b_ref, out_ref):
    neigh_mean = jnp.sum(neigh_ref[...], axis=1) * (1.0 / K)
    t = jnp.dot(self_ref[...], wts_ref[...], preferred_element_type=jnp.float32)
    t = t + jnp.dot(neigh_mean, wtn_ref[...], preferred_element_type=jnp.float32)
    t = t + b_ref[...]
    c = jnp.maximum(t, 0.0)
    norm2 = jnp.sum(c * c, axis=1, keepdims=True)
    out_ref[...] = c * jax.lax.rsqrt(jnp.maximum(norm2, 1e-24))


def _tail_body(self_ref, scsum_ref, wts_ref, wtn_ref, b_ref, out_ref):
    neigh_mean = scsum_ref[...] * (1.0 / K)
    t = jnp.dot(self_ref[...], wts_ref[...], preferred_element_type=jnp.float32)
    t = t + jnp.dot(neigh_mean, wtn_ref[...], preferred_element_type=jnp.float32)
    t = t + b_ref[...]
    c = jnp.maximum(t, 0.0)
    norm2 = jnp.sum(c * c, axis=1, keepdims=True)
    out_ref[...] = c * jax.lax.rsqrt(jnp.maximum(norm2, 1e-24))


def _sc_body(neigh_hbm, out_hbm, buf, obuf, sem_in, sem_out):
    ci = lax.axis_index("c")
    si = lax.axis_index("s")
    w = si * SC_NC + ci
    nj = (NCH_G - 1 - w) // SC_NW + 1

    def step(j, carry):
        chunk = w + j * SC_NW
        row = R_TC + chunk * CH
        pltpu.async_copy(neigh_hbm.at[pl.ds(row, CH)], buf, sem_in).wait()
        for i in range(CH):
            def kbody(k8, accs):
                new = list(accs)
                for kk in range(8):
                    kidx = k8 * 8 + kk
                    for h in range(HREG):
                        new[h] = new[h] + buf[i, kidx, pl.ds(h * LANES, LANES)]
                return tuple(new)
            accs = lax.fori_loop(
                0, K // 8, kbody,
                tuple(jnp.zeros((LANES,), jnp.float32) for _ in range(HREG)),
            )
            for h in range(HREG):
                obuf[i, pl.ds(h * LANES, LANES)] = accs[h]
        pltpu.async_copy(obuf, out_hbm.at[pl.ds(chunk * CH, CH)], sem_out).wait()
        return carry

    lax.fori_loop(0, nj, step, 0)


_sc_neigh_sum = functools.partial(
    pl.kernel,
    out_type=jax.ShapeDtypeStruct((R_SC, D_IN), jnp.float32),
    mesh=plsc.VectorSubcoreMesh(
        core_axis_name="c", subcore_axis_name="s",
        num_cores=SC_NC, num_subcores=SC_NS,
    ),
    scratch_types=[
        pltpu.VMEM((CH, K, D_IN), jnp.float32),
        pltpu.VMEM((CH, D_IN), jnp.float32),
        pltpu.SemaphoreType.DMA,
        pltpu.SemaphoreType.DMA,
    ],
)(_sc_body)


def kernel(self_embs, neigh_embs, W_self, b_self, W_neigh, b_neigh):
    wts = W_self.T
    wtn = W_neigh.T
    b = (b_self + b_neigh).reshape(1, D_OUT)
    sc_sum = _sc_neigh_sum(neigh_embs)

    head = pl.pallas_call(
        _head_body,
        grid=(R_TC // BN,),
        in_specs=[
            pl.BlockSpec((BN, D_IN), lambda i: (i, 0)),
            pl.BlockSpec((BN, K, D_IN), lambda i: (i, 0, 0)),
            pl.BlockSpec((D_IN, D_OUT), lambda i: (0, 0)),
            pl.BlockSpec((D_IN, D_OUT), lambda i: (0, 0)),
            pl.BlockSpec((1, D_OUT), lambda i: (0, 0)),
        ],
        out_specs=pl.BlockSpec((BN, D_OUT), lambda i: (i, 0)),
        out_shape=jax.ShapeDtypeStruct((R_TC, D_OUT), jnp.float32),
        compiler_params=pltpu.CompilerParams(
            dimension_semantics=("arbitrary",),
        ),
    )(self_embs, neigh_embs, wts, wtn, b)

    nhead = R_TC // BN
    tail = pl.pallas_call(
        _tail_body,
        grid=(R_SC // BN,),
        in_specs=[
            pl.BlockSpec((BN, D_IN), lambda i: (i + nhead, 0)),
            pl.BlockSpec((BN, D_IN), lambda i: (i, 0)),
            pl.BlockSpec((D_IN, D_OUT), lambda i: (0, 0)),
            pl.BlockSpec((D_IN, D_OUT), lambda i: (0, 0)),
            pl.BlockSpec((1, D_OUT), lambda i: (0, 0)),
        ],
        out_specs=pl.BlockSpec((BN, D_OUT), lambda i: (i, 0)),
        out_shape=jax.ShapeDtypeStruct((R_SC, D_OUT), jnp.float32),
        compiler_params=pltpu.CompilerParams(
            dimension_semantics=("arbitrary",),
        ),
    )(self_embs, sc_sum, wts, wtn, b)

    return jnp.concatenate([head, tail], axis=0)
